# baseline (speedup 1.0000x reference)
import functools

import jax
import jax.numpy as jnp
from jax import lax
from jax.experimental import pallas as pl
from jax.experimental.pallas import tpu as pltpu
from jax.experimental.pallas import tpu_sc as plsc

VOCAB = 1_000_000
EMBED_DIM = 32
BATCH = 16384
HIST = 20
IN_DIM = HIST * EMBED_DIM
HIDDEN = 256
OUT_DIM = 128

N_ROWS = BATCH * HIST
NUM_CORES = 2
NUM_SUBCORES = 16
NW = NUM_CORES * NUM_SUBCORES
PER_W = N_ROWS // NW
CHUNK = 2560
NCHUNK = PER_W // CHUNK

_sc_mesh = plsc.VectorSubcoreMesh(core_axis_name="c", subcore_axis_name="s")

NCT_FULL = (VOCAB // 128)
EDGE_COLS = VOCAB - NCT_FULL * 128
SLOTS = -(-NCT_FULL // NW)


@functools.partial(
    pl.kernel,
    mesh=_sc_mesh,
    out_type=jax.ShapeDtypeStruct((VOCAB * EMBED_DIM // 128, 128), jnp.float32),
    scratch_types=[
        pltpu.VMEM((EMBED_DIM, 128), jnp.float32),
        pltpu.VMEM((32, 128), jnp.float32),
        pltpu.VMEM((EMBED_DIM, EDGE_COLS), jnp.float32),
        pltpu.VMEM((16, 128), jnp.float32),
    ],
    compiler_params=pltpu.CompilerParams(
        use_tc_tiling_on_sc=True, needs_layout_passes=False),
)
def _sc_relayout(wvt_hbm, out_hbm, in_v, out_v, ine_v, oute_v):
    wid = lax.axis_index("s") * NUM_CORES + lax.axis_index("c")
    iota = lax.iota(jnp.int32, 16)

    def shuffle(src_ref, dst_ref, rows):
        for r in range(rows):
            for m in range(8):
                cvec = jnp.full((16,), 4 * r + m // 2, jnp.int32)
                g = plsc.load_gather(src_ref, [iota + 16 * (m % 2), cvec])
                dst_ref[r, pl.ds(16 * m, 16)] = g

    def body(j, carry):
        ct = wid + NW * j

        @pl.when(ct < NCT_FULL)
        def _():
            pltpu.sync_copy(wvt_hbm.at[:, pl.ds(ct * 128, 128)], in_v)
            shuffle(in_v, out_v, 32)
            pltpu.sync_copy(out_v, out_hbm.at[pl.ds(ct * 32, 32)])

        return carry

    lax.fori_loop(0, SLOTS, body, 0)

    @pl.when(wid == NW - 1)
    def _():
        pltpu.sync_copy(wvt_hbm.at[:, pl.ds(NCT_FULL * 128, EDGE_COLS)], ine_v)
        shuffle(ine_v, oute_v, 16)
        pltpu.sync_copy(oute_v, out_hbm.at[pl.ds(NCT_FULL * 32, 16)])


@functools.partial(
    pl.kernel,
    mesh=_sc_mesh,
    out_type=jax.ShapeDtypeStruct((N_ROWS, EMBED_DIM), jnp.float32),
    scratch_types=[
        pltpu.VMEM((CHUNK,), jnp.int32),
        pltpu.VMEM((CHUNK, EMBED_DIM), jnp.float32),
        pltpu.SemaphoreType.DMA,
    ],
    compiler_params=pltpu.CompilerParams(use_tc_tiling_on_sc=False),
)
def _sc_gather(idx_hbm, table_hbm, out_hbm, idx_v, rows_v, sem):
    wid = lax.axis_index("s") * NUM_CORES + lax.axis_index("c")
    base = wid * PER_W

    def body(i, carry):
        off = base + i * CHUNK
        pltpu.sync_copy(idx_hbm.at[pl.ds(off, CHUNK)], idx_v)
        pltpu.async_copy(table_hbm.at[idx_v], rows_v, sem).wait()
        pltpu.sync_copy(rows_v, out_hbm.at[pl.ds(off, CHUNK)])
        return carry

    lax.fori_loop(0, NCHUNK, body, 0)


BB = 1024
TB = BB // 8
NS = IN_DIM // 128


def _mlp_body(x_ref, w1_ref, b1_ref, w2_ref, o_ref):
    acc = jnp.zeros((BB, HIDDEN), dtype=jnp.float32)
    for ct in range(NS):
        xc = x_ref[:, ct, :, :].reshape(BB, 128)
        acc = acc + jnp.dot(xc, w1_ref[ct], preferred_element_type=jnp.float32)
    z = acc + b1_ref[...]
    h = 1.0 / (1.0 + jnp.exp(-z))
    o_ref[...] = jnp.dot(h, w2_ref[...], preferred_element_type=jnp.float32)


_mlp = pl.pallas_call(
    _mlp_body,
    grid=(BATCH // BB,),
    in_specs=[
        pl.BlockSpec((TB, NS, 8, 128), lambda i: (i, 0, 0, 0)),
        pl.BlockSpec((NS, 128, HIDDEN), lambda i: (0, 0, 0)),
        pl.BlockSpec((1, HIDDEN), lambda i: (0, 0)),
        pl.BlockSpec((HIDDEN, OUT_DIM), lambda i: (0, 0)),
    ],
    out_specs=pl.BlockSpec((BB, OUT_DIM), lambda i: (i, 0)),
    out_shape=jax.ShapeDtypeStruct((BATCH, OUT_DIM), jnp.float32),
)


def kernel(x, word_vectors, W1, b1, W2):
    idxp = (x.astype(jnp.int32)
             .reshape(BATCH // 8, 8, NS, 4)
             .transpose(0, 2, 1, 3)
             .reshape(-1))
    wv_lin = _sc_relayout(word_vectors.T)
    emb = _sc_gather(idxp, wv_lin.reshape(VOCAB, EMBED_DIM))
    e4d = emb.reshape(BATCH // 8, NS, 8, 128)
    w1v = W1.reshape(NS, 128, HIDDEN)
    return _mlp(e4d, w1v, b1.reshape(1, HIDDEN), W2)


# speedup vs baseline: 1.3148x; 1.3148x over previous
import functools

import jax
import jax.numpy as jnp
from jax import lax
from jax.experimental import pallas as pl
from jax.experimental.pallas import tpu as pltpu
from jax.experimental.pallas import tpu_sc as plsc

VOCAB = 1_000_000
EMBED_DIM = 32
BATCH = 16384
HIST = 20
IN_DIM = HIST * EMBED_DIM
HIDDEN = 256
OUT_DIM = 128

N_ROWS = BATCH * HIST
NUM_CORES = 2
NUM_SUBCORES = 16
NW = NUM_CORES * NUM_SUBCORES
PER_W = N_ROWS // NW
CHUNK = 2560
NCHUNK = PER_W // CHUNK

_sc_mesh = plsc.VectorSubcoreMesh(core_axis_name="c", subcore_axis_name="s")

NCT_FULL = (VOCAB // 128)
EDGE_COLS = VOCAB - NCT_FULL * 128
K = 2
UNIT_COLS = K * 128
UNIT_ROWS = K * 32
NUNITS = NCT_FULL // K
PAIRS = 62


@functools.partial(
    pl.kernel,
    mesh=_sc_mesh,
    out_type=jax.ShapeDtypeStruct((VOCAB * EMBED_DIM // 128, 128), jnp.float32),
    scratch_types=[
        pltpu.VMEM((2, EMBED_DIM, UNIT_COLS), jnp.float32),
        pltpu.VMEM((2, UNIT_ROWS, 128), jnp.float32),
        pltpu.VMEM((EMBED_DIM, EDGE_COLS), jnp.float32),
        pltpu.VMEM((16, 128), jnp.float32),
        pltpu.SemaphoreType.DMA,
        pltpu.SemaphoreType.DMA,
        pltpu.SemaphoreType.DMA,
        pltpu.SemaphoreType.DMA,
    ],
    compiler_params=pltpu.CompilerParams(
        use_tc_tiling_on_sc=True, needs_layout_passes=False),
)
def _sc_relayout(wvt_hbm, out_hbm, in_v, out_v, ine_v, oute_v,
                 si0, si1, so0, so1):
    wid = lax.axis_index("s") * NUM_CORES + lax.axis_index("c")
    iota = lax.iota(jnp.int32, 16)
    sin = (si0, si1)
    sout = (so0, so1)

    def shuffle(src_ref, dst_ref, rows):
        def srow(r, carry):
            for m in range(8):
                cvec = jnp.full((16,), 0, jnp.int32) + (4 * r + m // 2)
                g = plsc.load_gather(src_ref, [iota + 16 * (m % 2), cvec])
                dst_ref[r, pl.ds(16 * m, 16)] = g
            return carry

        lax.fori_loop(0, rows, srow, 0)

    def in_desc(u, b):
        return (wvt_hbm.at[:, pl.ds(u * UNIT_COLS, UNIT_COLS)],
                in_v.at[b], sin[b])

    def out_desc(u, b):
        return (out_v.at[b], out_hbm.at[pl.ds(u * UNIT_ROWS, UNIT_ROWS)],
                sout[b])

    def start_in(j, b):
        u = wid + NW * j

        @pl.when(u < NUNITS)
        def _():
            pltpu.async_copy(*in_desc(u, b))

    def half(j, b):
        u = wid + NW * j

        @pl.when(u < NUNITS)
        def _():
            pltpu.make_async_copy(*in_desc(u, b)).wait()

            @pl.when(j >= 2)
            def _():
                pltpu.make_async_copy(*out_desc(u - 2 * NW, b)).wait()

            shuffle(in_v.at[b], out_v.at[b], UNIT_ROWS)
            pltpu.async_copy(*out_desc(u, b))

    start_in(0, 0)

    def pair(p, carry):
        j = 2 * p
        start_in(j + 1, 1)
        half(j, 0)
        start_in(j + 2, 0)
        half(j + 1, 1)
        return carry

    lax.fori_loop(0, PAIRS, pair, 0)

    _BASE = NUNITS - 2 * NW
    _OFF = (2 * NW - (_BASE % (2 * NW))) % (2 * NW)
    for b in (0, 1):
        ub = _BASE + ((wid + NW * b + _OFF) % (2 * NW))
        pltpu.make_async_copy(*out_desc(ub, b)).wait()

    @pl.when(wid == NW - 1)
    def _():
        pltpu.sync_copy(wvt_hbm.at[:, pl.ds(NCT_FULL * 128, EDGE_COLS)], ine_v)
        shuffle(ine_v, oute_v, 16)
        pltpu.sync_copy(oute_v, out_hbm.at[pl.ds(NCT_FULL * 32, 16)])


@functools.partial(
    pl.kernel,
    mesh=_sc_mesh,
    out_type=jax.ShapeDtypeStruct((N_ROWS, EMBED_DIM), jnp.float32),
    scratch_types=[
        pltpu.VMEM((CHUNK,), jnp.int32),
        pltpu.VMEM((CHUNK, EMBED_DIM), jnp.float32),
        pltpu.SemaphoreType.DMA,
    ],
    compiler_params=pltpu.CompilerParams(use_tc_tiling_on_sc=False),
)
def _sc_gather(idx_hbm, table_hbm, out_hbm, idx_v, rows_v, sem):
    wid = lax.axis_index("s") * NUM_CORES + lax.axis_index("c")
    base = wid * PER_W

    def body(i, carry):
        off = base + i * CHUNK
        pltpu.sync_copy(idx_hbm.at[pl.ds(off, CHUNK)], idx_v)
        pltpu.async_copy(table_hbm.at[idx_v], rows_v, sem).wait()
        pltpu.sync_copy(rows_v, out_hbm.at[pl.ds(off, CHUNK)])
        return carry

    lax.fori_loop(0, NCHUNK, body, 0)


BB = 1024
TB = BB // 8
NS = IN_DIM // 128


def _mlp_body(x_ref, w1_ref, b1_ref, w2_ref, o_ref):
    acc = jnp.zeros((BB, HIDDEN), dtype=jnp.float32)
    for ct in range(NS):
        xc = x_ref[:, ct, :, :].reshape(BB, 128)
        acc = acc + jnp.dot(xc, w1_ref[ct], preferred_element_type=jnp.float32)
    z = acc + b1_ref[...]
    h = 1.0 / (1.0 + jnp.exp(-z))
    o_ref[...] = jnp.dot(h, w2_ref[...], preferred_element_type=jnp.float32)


_mlp = pl.pallas_call(
    _mlp_body,
    grid=(BATCH // BB,),
    in_specs=[
        pl.BlockSpec((TB, NS, 8, 128), lambda i: (i, 0, 0, 0)),
        pl.BlockSpec((NS, 128, HIDDEN), lambda i: (0, 0, 0)),
        pl.BlockSpec((1, HIDDEN), lambda i: (0, 0)),
        pl.BlockSpec((HIDDEN, OUT_DIM), lambda i: (0, 0)),
    ],
    out_specs=pl.BlockSpec((BB, OUT_DIM), lambda i: (i, 0)),
    out_shape=jax.ShapeDtypeStruct((BATCH, OUT_DIM), jnp.float32),
)


def kernel(x, word_vectors, W1, b1, W2):
    idxp = (x.astype(jnp.int32)
             .reshape(BATCH // 8, 8, NS, 4)
             .transpose(0, 2, 1, 3)
             .reshape(-1))
    wv_lin = _sc_relayout(word_vectors.T)
    emb = _sc_gather(idxp, wv_lin.reshape(VOCAB, EMBED_DIM))
    e4d = emb.reshape(BATCH // 8, NS, 8, 128)
    w1v = W1.reshape(NS, 128, HIDDEN)
    return _mlp(e4d, w1v, b1.reshape(1, HIDDEN), W2)


# speedup vs baseline: 2.2543x; 1.7146x over previous
import functools

import jax
import jax.numpy as jnp
from jax import lax
from jax.experimental import pallas as pl
from jax.experimental.pallas import tpu as pltpu
from jax.experimental.pallas import tpu_sc as plsc

VOCAB = 1_000_000
EMBED_DIM = 32
BATCH = 16384
HIST = 20
IN_DIM = HIST * EMBED_DIM
HIDDEN = 256
OUT_DIM = 128

N_ROWS = BATCH * HIST
NUM_CORES = 2
NUM_SUBCORES = 16
NW = NUM_CORES * NUM_SUBCORES
PER_W = N_ROWS // NW
CHUNK = 2560
NCHUNK = PER_W // CHUNK

_sc_mesh = plsc.VectorSubcoreMesh(core_axis_name="c", subcore_axis_name="s")

NCT_FULL = (VOCAB // 128)
EDGE_COLS = VOCAB - NCT_FULL * 128
K = 2
UNIT_COLS = K * 128
UNIT_ROWS = K * 32
NUNITS = NCT_FULL // K
PAIRS = 62


@functools.partial(
    pl.kernel,
    mesh=_sc_mesh,
    out_type=jax.ShapeDtypeStruct((VOCAB * EMBED_DIM // 128, 128), jnp.float32),
    scratch_types=[
        pltpu.VMEM((2, EMBED_DIM, UNIT_COLS), jnp.float32),
        pltpu.VMEM((2, UNIT_ROWS, 128), jnp.float32),
        pltpu.VMEM((EMBED_DIM, EDGE_COLS), jnp.float32),
        pltpu.VMEM((16, 128), jnp.float32),
        pltpu.SemaphoreType.DMA,
        pltpu.SemaphoreType.DMA,
        pltpu.SemaphoreType.DMA,
        pltpu.SemaphoreType.DMA,
    ],
    compiler_params=pltpu.CompilerParams(
        use_tc_tiling_on_sc=True, needs_layout_passes=False),
)
def _sc_relayout(wvt_hbm, out_hbm, in_v, out_v, ine_v, oute_v,
                 si0, si1, so0, so1):
    wid = lax.axis_index("s") * NUM_CORES + lax.axis_index("c")
    iota = lax.iota(jnp.int32, 16)
    sin = (si0, si1)
    sout = (so0, so1)

    def shuffle(src_ref, dst_ref, rows):
        @plsc.parallel_loop(0, rows, unroll=4)
        def srow(r):
            for m in range(8):
                cvec = jnp.full((16,), 0, jnp.int32) + (4 * r + m // 2)
                g = plsc.load_gather(src_ref, [iota + 16 * (m % 2), cvec])
                dst_ref[r, pl.ds(16 * m, 16)] = g

    def in_desc(u, b):
        return (wvt_hbm.at[:, pl.ds(u * UNIT_COLS, UNIT_COLS)],
                in_v.at[b], sin[b])

    def out_desc(u, b):
        return (out_v.at[b], out_hbm.at[pl.ds(u * UNIT_ROWS, UNIT_ROWS)],
                sout[b])

    def start_in(j, b):
        u = wid + NW * j

        @pl.when(u < NUNITS)
        def _():
            pltpu.async_copy(*in_desc(u, b))

    def half(j, b):
        u = wid + NW * j

        @pl.when(u < NUNITS)
        def _():
            pltpu.make_async_copy(*in_desc(u, b)).wait()

            @pl.when(j >= 2)
            def _():
                pltpu.make_async_copy(*out_desc(u - 2 * NW, b)).wait()

            shuffle(in_v.at[b], out_v.at[b], UNIT_ROWS)
            pltpu.async_copy(*out_desc(u, b))

    start_in(0, 0)

    def pair(p, carry):
        j = 2 * p
        start_in(j + 1, 1)
        half(j, 0)
        start_in(j + 2, 0)
        half(j + 1, 1)
        return carry

    lax.fori_loop(0, PAIRS, pair, 0)

    _BASE = NUNITS - 2 * NW
    _OFF = (2 * NW - (_BASE % (2 * NW))) % (2 * NW)
    for b in (0, 1):
        ub = _BASE + ((wid + NW * b + _OFF) % (2 * NW))
        pltpu.make_async_copy(*out_desc(ub, b)).wait()

    @pl.when(wid == NW - 1)
    def _():
        pltpu.sync_copy(wvt_hbm.at[:, pl.ds(NCT_FULL * 128, EDGE_COLS)], ine_v)
        shuffle(ine_v, oute_v, 16)
        pltpu.sync_copy(oute_v, out_hbm.at[pl.ds(NCT_FULL * 32, 16)])


@functools.partial(
    pl.kernel,
    mesh=_sc_mesh,
    out_type=jax.ShapeDtypeStruct((N_ROWS, EMBED_DIM), jnp.float32),
    scratch_types=[
        pltpu.VMEM((CHUNK,), jnp.int32),
        pltpu.VMEM((CHUNK, EMBED_DIM), jnp.float32),
        pltpu.SemaphoreType.DMA,
    ],
    compiler_params=pltpu.CompilerParams(use_tc_tiling_on_sc=False),
)
def _sc_gather(idx_hbm, table_hbm, out_hbm, idx_v, rows_v, sem):
    wid = lax.axis_index("s") * NUM_CORES + lax.axis_index("c")
    base = wid * PER_W

    def body(i, carry):
        off = base + i * CHUNK
        pltpu.sync_copy(idx_hbm.at[pl.ds(off, CHUNK)], idx_v)
        pltpu.async_copy(table_hbm.at[idx_v], rows_v, sem).wait()
        pltpu.sync_copy(rows_v, out_hbm.at[pl.ds(off, CHUNK)])
        return carry

    lax.fori_loop(0, NCHUNK, body, 0)


BB = 1024
TB = BB // 8
NS = IN_DIM // 128


def _mlp_body(x_ref, w1_ref, b1_ref, w2_ref, o_ref):
    acc = jnp.zeros((BB, HIDDEN), dtype=jnp.float32)
    for ct in range(NS):
        xc = x_ref[:, ct, :, :].reshape(BB, 128)
        acc = acc + jnp.dot(xc, w1_ref[ct], preferred_element_type=jnp.float32)
    z = acc + b1_ref[...]
    h = 1.0 / (1.0 + jnp.exp(-z))
    o_ref[...] = jnp.dot(h, w2_ref[...], preferred_element_type=jnp.float32)


_mlp = pl.pallas_call(
    _mlp_body,
    grid=(BATCH // BB,),
    in_specs=[
        pl.BlockSpec((TB, NS, 8, 128), lambda i: (i, 0, 0, 0)),
        pl.BlockSpec((NS, 128, HIDDEN), lambda i: (0, 0, 0)),
        pl.BlockSpec((1, HIDDEN), lambda i: (0, 0)),
        pl.BlockSpec((HIDDEN, OUT_DIM), lambda i: (0, 0)),
    ],
    out_specs=pl.BlockSpec((BB, OUT_DIM), lambda i: (i, 0)),
    out_shape=jax.ShapeDtypeStruct((BATCH, OUT_DIM), jnp.float32),
)


def kernel(x, word_vectors, W1, b1, W2):
    idxp = (x.astype(jnp.int32)
             .reshape(BATCH // 8, 8, NS, 4)
             .transpose(0, 2, 1, 3)
             .reshape(-1))
    wv_lin = _sc_relayout(word_vectors.T)
    emb = _sc_gather(idxp, wv_lin.reshape(VOCAB, EMBED_DIM))
    e4d = emb.reshape(BATCH // 8, NS, 8, 128)
    w1v = W1.reshape(NS, 128, HIDDEN)
    return _mlp(e4d, w1v, b1.reshape(1, HIDDEN), W2)


# speedup vs baseline: 2.2561x; 1.0008x over previous
import functools

import jax
import jax.numpy as jnp
from jax import lax
from jax.experimental import pallas as pl
from jax.experimental.pallas import tpu as pltpu
from jax.experimental.pallas import tpu_sc as plsc

VOCAB = 1_000_000
EMBED_DIM = 32
BATCH = 16384
HIST = 20
IN_DIM = HIST * EMBED_DIM
HIDDEN = 256
OUT_DIM = 128

N_ROWS = BATCH * HIST
NUM_CORES = 2
NUM_SUBCORES = 16
NW = NUM_CORES * NUM_SUBCORES
PER_W = N_ROWS // NW
CHUNK = 2560
NCHUNK = PER_W // CHUNK

_sc_mesh = plsc.VectorSubcoreMesh(core_axis_name="c", subcore_axis_name="s")

NCT_FULL = (VOCAB // 128)
EDGE_COLS = VOCAB - NCT_FULL * 128
K = 2
UNIT_COLS = K * 128
UNIT_ROWS = K * 32
NUNITS = NCT_FULL // K
PAIRS = 62


@functools.partial(
    pl.kernel,
    mesh=_sc_mesh,
    out_type=jax.ShapeDtypeStruct((VOCAB * EMBED_DIM // 128, 128), jnp.float32),
    scratch_types=[
        pltpu.VMEM((2, EMBED_DIM, UNIT_COLS), jnp.float32),
        pltpu.VMEM((2, UNIT_ROWS, 128), jnp.float32),
        pltpu.VMEM((EMBED_DIM, EDGE_COLS), jnp.float32),
        pltpu.VMEM((16, 128), jnp.float32),
        pltpu.SemaphoreType.DMA,
        pltpu.SemaphoreType.DMA,
        pltpu.SemaphoreType.DMA,
        pltpu.SemaphoreType.DMA,
    ],
    compiler_params=pltpu.CompilerParams(
        use_tc_tiling_on_sc=True, needs_layout_passes=False),
)
def _sc_relayout(wvt_hbm, out_hbm, in_v, out_v, ine_v, oute_v,
                 si0, si1, so0, so1):
    wid = lax.axis_index("s") * NUM_CORES + lax.axis_index("c")
    iota = lax.iota(jnp.int32, 16)
    sin = (si0, si1)
    sout = (so0, so1)

    def shuffle(src_ref, dst_ref, rows):
        @plsc.parallel_loop(0, rows, unroll=8)
        def srow(r):
            for m in range(8):
                cvec = jnp.full((16,), 0, jnp.int32) + (4 * r + m // 2)
                g = plsc.load_gather(src_ref, [iota + 16 * (m % 2), cvec])
                dst_ref[r, pl.ds(16 * m, 16)] = g

    def in_desc(u, b):
        return (wvt_hbm.at[:, pl.ds(u * UNIT_COLS, UNIT_COLS)],
                in_v.at[b], sin[b])

    def out_desc(u, b):
        return (out_v.at[b], out_hbm.at[pl.ds(u * UNIT_ROWS, UNIT_ROWS)],
                sout[b])

    def start_in(j, b):
        u = wid + NW * j

        @pl.when(u < NUNITS)
        def _():
            pltpu.async_copy(*in_desc(u, b))

    def half(j, b):
        u = wid + NW * j

        @pl.when(u < NUNITS)
        def _():
            pltpu.make_async_copy(*in_desc(u, b)).wait()

            @pl.when(j >= 2)
            def _():
                pltpu.make_async_copy(*out_desc(u - 2 * NW, b)).wait()

            shuffle(in_v.at[b], out_v.at[b], UNIT_ROWS)
            pltpu.async_copy(*out_desc(u, b))

    start_in(0, 0)

    def pair(p, carry):
        j = 2 * p
        start_in(j + 1, 1)
        half(j, 0)
        start_in(j + 2, 0)
        half(j + 1, 1)
        return carry

    lax.fori_loop(0, PAIRS, pair, 0)

    _BASE = NUNITS - 2 * NW
    _OFF = (2 * NW - (_BASE % (2 * NW))) % (2 * NW)
    for b in (0, 1):
        ub = _BASE + ((wid + NW * b + _OFF) % (2 * NW))
        pltpu.make_async_copy(*out_desc(ub, b)).wait()

    @pl.when(wid == NW - 1)
    def _():
        pltpu.sync_copy(wvt_hbm.at[:, pl.ds(NCT_FULL * 128, EDGE_COLS)], ine_v)
        shuffle(ine_v, oute_v, 16)
        pltpu.sync_copy(oute_v, out_hbm.at[pl.ds(NCT_FULL * 32, 16)])


@functools.partial(
    pl.kernel,
    mesh=_sc_mesh,
    out_type=jax.ShapeDtypeStruct((N_ROWS, EMBED_DIM), jnp.float32),
    scratch_types=[
        pltpu.VMEM((CHUNK,), jnp.int32),
        pltpu.VMEM((CHUNK, EMBED_DIM), jnp.float32),
        pltpu.SemaphoreType.DMA,
    ],
    compiler_params=pltpu.CompilerParams(use_tc_tiling_on_sc=False),
)
def _sc_gather(idx_hbm, table_hbm, out_hbm, idx_v, rows_v, sem):
    wid = lax.axis_index("s") * NUM_CORES + lax.axis_index("c")
    base = wid * PER_W

    def body(i, carry):
        off = base + i * CHUNK
        pltpu.sync_copy(idx_hbm.at[pl.ds(off, CHUNK)], idx_v)
        pltpu.async_copy(table_hbm.at[idx_v], rows_v, sem).wait()
        pltpu.sync_copy(rows_v, out_hbm.at[pl.ds(off, CHUNK)])
        return carry

    lax.fori_loop(0, NCHUNK, body, 0)


BB = 1024
TB = BB // 8
NS = IN_DIM // 128


def _mlp_body(x_ref, w1_ref, b1_ref, w2_ref, o_ref):
    acc = jnp.zeros((BB, HIDDEN), dtype=jnp.float32)
    for ct in range(NS):
        xc = x_ref[:, ct, :, :].reshape(BB, 128)
        acc = acc + jnp.dot(xc, w1_ref[ct], preferred_element_type=jnp.float32)
    z = acc + b1_ref[...]
    h = 1.0 / (1.0 + jnp.exp(-z))
    o_ref[...] = jnp.dot(h, w2_ref[...], preferred_element_type=jnp.float32)


_mlp = pl.pallas_call(
    _mlp_body,
    grid=(BATCH // BB,),
    in_specs=[
        pl.BlockSpec((TB, NS, 8, 128), lambda i: (i, 0, 0, 0)),
        pl.BlockSpec((NS, 128, HIDDEN), lambda i: (0, 0, 0)),
        pl.BlockSpec((1, HIDDEN), lambda i: (0, 0)),
        pl.BlockSpec((HIDDEN, OUT_DIM), lambda i: (0, 0)),
    ],
    out_specs=pl.BlockSpec((BB, OUT_DIM), lambda i: (i, 0)),
    out_shape=jax.ShapeDtypeStruct((BATCH, OUT_DIM), jnp.float32),
)


def kernel(x, word_vectors, W1, b1, W2):
    idxp = (x.astype(jnp.int32)
             .reshape(BATCH // 8, 8, NS, 4)
             .transpose(0, 2, 1, 3)
             .reshape(-1))
    wv_lin = _sc_relayout(word_vectors.T)
    emb = _sc_gather(idxp, wv_lin.reshape(VOCAB, EMBED_DIM))
    e4d = emb.reshape(BATCH // 8, NS, 8, 128)
    w1v = W1.reshape(NS, 128, HIDDEN)
    return _mlp(e4d, w1v, b1.reshape(1, HIDDEN), W2)
